# baseline (speedup 1.0000x reference)
import functools

import jax
import jax.numpy as jnp
from jax import lax
from jax.experimental import pallas as pl
from jax.experimental.pallas import tpu as pltpu
from jax.experimental.pallas import tpu_sc as plsc

NC = 2
NS = 16
NW = NC * NS
LANES = 16

CH = 128
RB = 10240


def _round_up(a, b):
  return ((a + b - 1) // b) * b


def _sc_mesh():
  return plsc.VectorSubcoreMesh(core_axis_name="c", subcore_axis_name="s")


def _make_deg_kernel(n_chunks, n_pad):
  @functools.partial(
      pl.kernel,
      out_type=jax.ShapeDtypeStruct((2, NW, n_pad // CH, CH), jnp.float32),
      mesh=_sc_mesh(),
      compiler_params=pltpu.CompilerParams(needs_layout_passes=False),
      scratch_types=[
          pltpu.VMEM((n_chunks, CH), jnp.int32),
          pltpu.VMEM((n_pad // CH, CH), jnp.float32),
          pltpu.VMEM((n_pad // CH, CH), jnp.float32),
      ],
  )
  def deg_kernel(pk_hbm, out_hbm, pk_v, ho_v, hi_v):
    cid = lax.axis_index("c")
    sid = lax.axis_index("s")
    wid = sid * NC + cid
    base = wid * n_chunks
    pltpu.sync_copy(pk_hbm.at[pl.ds(base, n_chunks)], pk_v)

    zero = jnp.zeros((LANES,), jnp.float32)

    @pl.loop(0, n_pad // CH)
    def _(i):
      for jj in range(CH // LANES):
        ho_v[i, pl.ds(jj * LANES, LANES)] = zero
        hi_v[i, pl.ds(jj * LANES, LANES)] = zero

    ones = jnp.ones((LANES,), jnp.float32)

    @pl.loop(0, n_chunks)
    def _(j):
      for jj in range(CH // LANES):
        pk = pk_v[j, pl.ds(jj * LANES, LANES)]
        s_idx = pk & 0xFFFF
        d_idx = pk >> 16
        plsc.addupdate_scatter(ho_v, [s_idx // CH, s_idx % CH], ones)
        plsc.addupdate_scatter(hi_v, [d_idx // CH, d_idx % CH], ones)

    pltpu.sync_copy(ho_v, out_hbm.at[0, wid])
    pltpu.sync_copy(hi_v, out_hbm.at[1, wid])

  return deg_kernel


SD = 2


def _make_agg_kernel(d, e_w, n_pad, ch, nbuf, n_phases):
  rows_per_tile = n_pad // NS
  n_chunks = e_w // ch
  assert n_chunks % (nbuf * n_phases) == 0
  cpp = n_chunks // n_phases
  groups = cpp // nbuf
  pf = nbuf - SD
  assert groups >= 2 and pf >= 1

  @functools.partial(
      pl.kernel,
      out_type=jax.ShapeDtypeStruct((NC, n_pad, d), jnp.float32),
      mesh=_sc_mesh(),
      compiler_params=pltpu.CompilerParams(
          needs_layout_passes=False, use_tc_tiling_on_sc=False),
      scratch_types=[
          pltpu.VMEM((cpp, ch), jnp.int32),
          pltpu.VMEM((cpp, ch), jnp.int32),
          pltpu.VMEM((nbuf * ch, d), jnp.float32),
          pltpu.VMEM_SHARED((n_pad, d), jnp.float32),
          [pltpu.SemaphoreType.DMA] * nbuf,
          [pltpu.SemaphoreType.DMA] * nbuf,
      ],
  )
  def agg_kernel(table_hbm, src_hbm, dst_hbm, out_hbm, src_v, dst_v, gbuf,
                 acc, gsems, ssems):
    cid = lax.axis_index("c")
    sid = lax.axis_index("s")
    wid = sid * NC + cid

    def buf(b):
      return gbuf.at[pl.ds(b * ch, ch)]

    def g_issue(j, b):
      pltpu.async_copy(table_hbm.at[src_v.at[j]], buf(b), gsems[b])

    def g_wait(j, b):
      pltpu.make_async_copy(table_hbm.at[src_v.at[j]], buf(b), gsems[b]).wait()

    def s_issue(j, b):
      pltpu.async_copy(buf(b), acc.at[dst_v.at[j]], ssems[b], add=True)

    def s_wait(j, b):
      pltpu.make_async_copy(buf(b), acc.at[dst_v.at[j]], ssems[b]).wait()

    zero = jnp.zeros((LANES,), jnp.float32)

    @pl.loop(0, ch)
    def _(r):
      for jj in range(d // LANES):
        gbuf[r, pl.ds(jj * LANES, LANES)] = zero

    r0 = sid * rows_per_tile
    for k in range(rows_per_tile // ch):
      pltpu.sync_copy(buf(0), acc.at[pl.ds(r0 + k * ch, ch)])
    plsc.subcore_barrier()

    for ph in range(n_phases):
      base = wid * n_chunks + ph * cpp
      pltpu.sync_copy(src_hbm.at[pl.ds(base, cpp)], src_v)
      pltpu.sync_copy(dst_hbm.at[pl.ds(base, cpp)], dst_v)

      for b in range(pf):
        g_issue(b, b)

      def step(p, b, do_swait, do_gissue):
        if do_swait:
          s_wait(p - SD, (b - SD) % nbuf)
        if do_gissue:
          g_issue(p + pf, (b + pf) % nbuf)
        g_wait(p, b)
        s_issue(p, b)

      for b in range(nbuf):
        step(b, b, b >= SD, b + pf <= cpp - 1)

      @pl.loop(1, groups - 1)
      def _(g):
        j0 = g * nbuf
        for b in range(nbuf):
          step(j0 + b, b, True, True)

      j0 = (groups - 1) * nbuf
      for b in range(nbuf):
        step(j0 + b, b, True, b <= nbuf - 1 - pf)
      for k in range(SD):
        s_wait(cpp - SD + k, (cpp - SD + k) % nbuf)

    plsc.subcore_barrier()
    pltpu.sync_copy(acc.at[pl.ds(r0, rows_per_tile)],
                    out_hbm.at[cid, pl.ds(r0, rows_per_tile)])

  return agg_kernel


def _prep1_body(dego_ref, degi_ref, x_ref, w0_ref, h_ref, ain_ref, aout_ref):
  dego = jnp.sum(dego_ref[...], axis=0)
  degi = jnp.sum(degi_ref[...], axis=0)
  a_out = lax.rsqrt(jnp.maximum(dego, 1.0))
  a_in = lax.rsqrt(jnp.maximum(degi, 1.0))
  h = jnp.dot(x_ref[...], w0_ref[...], preferred_element_type=jnp.float32)
  h_ref[...] = h * a_out[:, None]
  ain_ref[...] = a_in[:, None]
  aout_ref[...] = a_out[:, None]


def _mid_body(p_ref, ain_ref, aout_ref, w1_ref, out_ref):
  agg = p_ref[0] + p_ref[1]
  h = jnp.maximum(agg * ain_ref[...], 0.0)
  h2 = jnp.dot(h, w1_ref[...], preferred_element_type=jnp.float32)
  out_ref[...] = h2 * aout_ref[...]


def _final_body(p_ref, ain_ref, out_ref):
  s = (p_ref[0] + p_ref[1]) * ain_ref[...]
  m = jnp.max(s, axis=-1, keepdims=True)
  e = jnp.exp(s - m)
  out_ref[...] = e / jnp.sum(e, axis=-1, keepdims=True)


def kernel(x, edge_index, W0, W1):
  n, d_in = x.shape
  d_hid = W0.shape[1]
  d_out = W1.shape[1]
  e = edge_index.shape[1]

  n_pad = _round_up(n + 1, NS * CH)
  e_w = _round_up(e, NW * CH * 8) // NW
  e_pad = e_w * NW
  n_chunks = e_w // CH

  src = edge_index[0]
  dst = edge_index[1]
  pad_e = e_pad - e
  src_p = jnp.concatenate([src, jnp.full((pad_e,), n, jnp.int32)])
  dst_p = jnp.concatenate([dst, jnp.full((pad_e,), n, jnp.int32)])
  pk2d = ((dst_p << 16) | src_p).reshape(e_pad // CH, CH)
  x_pad = jnp.zeros((n_pad, d_in), x.dtype).at[:n].set(x)

  deg_kernel = _make_deg_kernel(n_chunks, n_pad)
  deg_p = deg_kernel(pk2d).reshape(2, NW, n_pad)

  grid = n_pad // RB
  h1, a_in, a_out = pl.pallas_call(
      _prep1_body,
      grid=(grid,),
      in_specs=[
          pl.BlockSpec((NW, RB), lambda i: (0, i)),
          pl.BlockSpec((NW, RB), lambda i: (0, i)),
          pl.BlockSpec((RB, d_in), lambda i: (i, 0)),
          pl.BlockSpec((d_in, d_hid), lambda i: (0, 0)),
      ],
      out_specs=[
          pl.BlockSpec((RB, d_hid), lambda i: (i, 0)),
          pl.BlockSpec((RB, 1), lambda i: (i, 0)),
          pl.BlockSpec((RB, 1), lambda i: (i, 0)),
      ],
      out_shape=[
          jax.ShapeDtypeStruct((n_pad, d_hid), jnp.float32),
          jax.ShapeDtypeStruct((n_pad, 1), jnp.float32),
          jax.ShapeDtypeStruct((n_pad, 1), jnp.float32),
      ],
  )(deg_p[0], deg_p[1], x_pad, W0)

  src64 = src_p.reshape(e_pad // 64, 64)
  dst64 = dst_p.reshape(e_pad // 64, 64)
  agg_h = _make_agg_kernel(d_hid, e_w, n_pad, 64, 4, 2)
  p1 = agg_h(h1, src64, dst64)

  h2 = pl.pallas_call(
      _mid_body,
      grid=(grid,),
      in_specs=[
          pl.BlockSpec((NC, RB, d_hid), lambda i: (0, i, 0)),
          pl.BlockSpec((RB, 1), lambda i: (i, 0)),
          pl.BlockSpec((RB, 1), lambda i: (i, 0)),
          pl.BlockSpec((d_hid, d_out), lambda i: (0, 0)),
      ],
      out_specs=pl.BlockSpec((RB, d_out), lambda i: (i, 0)),
      out_shape=jax.ShapeDtypeStruct((n_pad, d_out), jnp.float32),
  )(p1, a_in, a_out, W1)

  src128 = src_p.reshape(e_pad // CH, CH)
  dst128 = dst_p.reshape(e_pad // CH, CH)
  agg_o = _make_agg_kernel(d_out, e_w, n_pad, CH, 8, 1)
  p2 = agg_o(h2, src128, dst128)

  out = pl.pallas_call(
      _final_body,
      grid=(grid,),
      in_specs=[
          pl.BlockSpec((NC, RB, d_out), lambda i: (0, i, 0)),
          pl.BlockSpec((RB, 1), lambda i: (i, 0)),
      ],
      out_specs=pl.BlockSpec((RB, d_out), lambda i: (i, 0)),
      out_shape=jax.ShapeDtypeStruct((n_pad, d_out), jnp.float32),
  )(p2, a_in)

  return out[:n]


# speedup vs baseline: 1.0017x; 1.0017x over previous
import functools

import jax
import jax.numpy as jnp
from jax import lax
from jax.experimental import pallas as pl
from jax.experimental.pallas import tpu as pltpu
from jax.experimental.pallas import tpu_sc as plsc

NC = 2
NS = 16
NW = NC * NS
LANES = 16

CH = 128
RB = 2048


def _round_up(a, b):
  return ((a + b - 1) // b) * b


def _sc_mesh():
  return plsc.VectorSubcoreMesh(core_axis_name="c", subcore_axis_name="s")


def _make_deg_kernel(n_chunks, n_pad):
  @functools.partial(
      pl.kernel,
      out_type=jax.ShapeDtypeStruct((2, NW, n_pad // CH, CH), jnp.float32),
      mesh=_sc_mesh(),
      compiler_params=pltpu.CompilerParams(needs_layout_passes=False),
      scratch_types=[
          pltpu.VMEM((n_chunks, CH), jnp.int32),
          pltpu.VMEM((n_pad // CH, CH), jnp.float32),
          pltpu.VMEM((n_pad // CH, CH), jnp.float32),
      ],
  )
  def deg_kernel(pk_hbm, out_hbm, pk_v, ho_v, hi_v):
    cid = lax.axis_index("c")
    sid = lax.axis_index("s")
    wid = sid * NC + cid
    base = wid * n_chunks
    pltpu.sync_copy(pk_hbm.at[pl.ds(base, n_chunks)], pk_v)

    zero = jnp.zeros((LANES,), jnp.float32)

    @pl.loop(0, n_pad // CH)
    def _(i):
      for jj in range(CH // LANES):
        ho_v[i, pl.ds(jj * LANES, LANES)] = zero
        hi_v[i, pl.ds(jj * LANES, LANES)] = zero

    ones = jnp.ones((LANES,), jnp.float32)

    @pl.loop(0, n_chunks)
    def _(j):
      for jj in range(CH // LANES):
        pk = pk_v[j, pl.ds(jj * LANES, LANES)]
        s_idx = pk & 0xFFFF
        d_idx = pk >> 16
        plsc.addupdate_scatter(ho_v, [s_idx // CH, s_idx % CH], ones)
        plsc.addupdate_scatter(hi_v, [d_idx // CH, d_idx % CH], ones)

    pltpu.sync_copy(ho_v, out_hbm.at[0, wid])
    pltpu.sync_copy(hi_v, out_hbm.at[1, wid])

  return deg_kernel


SD = 2


def _make_agg_kernel(d, e_w, n_pad, ch, nbuf, n_phases):
  rows_per_tile = n_pad // NS
  n_chunks = e_w // ch
  assert n_chunks % (nbuf * n_phases) == 0
  cpp = n_chunks // n_phases
  groups = cpp // nbuf
  pf = nbuf - SD
  assert groups >= 2 and pf >= 1

  @functools.partial(
      pl.kernel,
      out_type=jax.ShapeDtypeStruct((NC, n_pad, d), jnp.float32),
      mesh=_sc_mesh(),
      compiler_params=pltpu.CompilerParams(
          needs_layout_passes=False, use_tc_tiling_on_sc=False),
      scratch_types=[
          pltpu.VMEM((cpp, ch), jnp.int32),
          pltpu.VMEM((cpp, ch), jnp.int32),
          pltpu.VMEM((nbuf * ch, d), jnp.float32),
          pltpu.VMEM_SHARED((n_pad, d), jnp.float32),
          [pltpu.SemaphoreType.DMA] * nbuf,
          [pltpu.SemaphoreType.DMA] * nbuf,
      ],
  )
  def agg_kernel(table_hbm, src_hbm, dst_hbm, out_hbm, src_v, dst_v, gbuf,
                 acc, gsems, ssems):
    cid = lax.axis_index("c")
    sid = lax.axis_index("s")
    wid = sid * NC + cid

    def buf(b):
      return gbuf.at[pl.ds(b * ch, ch)]

    def g_issue(j, b):
      pltpu.async_copy(table_hbm.at[src_v.at[j]], buf(b), gsems[b])

    def g_wait(j, b):
      pltpu.make_async_copy(table_hbm.at[src_v.at[j]], buf(b), gsems[b]).wait()

    def s_issue(j, b):
      pltpu.async_copy(buf(b), acc.at[dst_v.at[j]], ssems[b], add=True)

    def s_wait(j, b):
      pltpu.make_async_copy(buf(b), acc.at[dst_v.at[j]], ssems[b]).wait()

    zero = jnp.zeros((LANES,), jnp.float32)

    @pl.loop(0, ch)
    def _(r):
      for jj in range(d // LANES):
        gbuf[r, pl.ds(jj * LANES, LANES)] = zero

    r0 = sid * rows_per_tile
    for k in range(rows_per_tile // ch):
      pltpu.sync_copy(buf(0), acc.at[pl.ds(r0 + k * ch, ch)])
    plsc.subcore_barrier()

    for ph in range(n_phases):
      base = wid * n_chunks + ph * cpp
      pltpu.sync_copy(src_hbm.at[pl.ds(base, cpp)], src_v)
      pltpu.sync_copy(dst_hbm.at[pl.ds(base, cpp)], dst_v)

      for b in range(pf):
        g_issue(b, b)

      def step(p, b, do_swait, do_gissue):
        if do_swait:
          s_wait(p - SD, (b - SD) % nbuf)
        if do_gissue:
          g_issue(p + pf, (b + pf) % nbuf)
        g_wait(p, b)
        s_issue(p, b)

      for b in range(nbuf):
        step(b, b, b >= SD, b + pf <= cpp - 1)

      @pl.loop(1, groups - 1)
      def _(g):
        j0 = g * nbuf
        for b in range(nbuf):
          step(j0 + b, b, True, True)

      j0 = (groups - 1) * nbuf
      for b in range(nbuf):
        step(j0 + b, b, True, b <= nbuf - 1 - pf)
      for k in range(SD):
        s_wait(cpp - SD + k, (cpp - SD + k) % nbuf)

    plsc.subcore_barrier()
    pltpu.sync_copy(acc.at[pl.ds(r0, rows_per_tile)],
                    out_hbm.at[cid, pl.ds(r0, rows_per_tile)])

  return agg_kernel


def _prep1_body(dego_ref, degi_ref, x_ref, w0_ref, h_ref, ain_ref, aout_ref):
  dego = jnp.sum(dego_ref[...], axis=0)
  degi = jnp.sum(degi_ref[...], axis=0)
  a_out = lax.rsqrt(jnp.maximum(dego, 1.0))
  a_in = lax.rsqrt(jnp.maximum(degi, 1.0))
  h = jnp.dot(x_ref[...], w0_ref[...], preferred_element_type=jnp.float32)
  h_ref[...] = h * a_out[:, None]
  ain_ref[...] = a_in[:, None]
  aout_ref[...] = a_out[:, None]


def _mid_body(p_ref, ain_ref, aout_ref, w1_ref, out_ref):
  agg = p_ref[0] + p_ref[1]
  h = jnp.maximum(agg * ain_ref[...], 0.0)
  h2 = jnp.dot(h, w1_ref[...], preferred_element_type=jnp.float32)
  out_ref[...] = h2 * aout_ref[...]


def _final_body(p_ref, ain_ref, out_ref):
  s = (p_ref[0] + p_ref[1]) * ain_ref[...]
  m = jnp.max(s, axis=-1, keepdims=True)
  e = jnp.exp(s - m)
  out_ref[...] = e / jnp.sum(e, axis=-1, keepdims=True)


def kernel(x, edge_index, W0, W1):
  n, d_in = x.shape
  d_hid = W0.shape[1]
  d_out = W1.shape[1]
  e = edge_index.shape[1]

  n_pad = _round_up(n + 1, NS * CH)
  e_w = _round_up(e, NW * CH * 8) // NW
  e_pad = e_w * NW
  n_chunks = e_w // CH

  src = edge_index[0]
  dst = edge_index[1]
  pad_e = e_pad - e
  src_p = jnp.concatenate([src, jnp.full((pad_e,), n, jnp.int32)])
  dst_p = jnp.concatenate([dst, jnp.full((pad_e,), n, jnp.int32)])
  pk2d = ((dst_p << 16) | src_p).reshape(e_pad // CH, CH)
  x_pad = jnp.zeros((n_pad, d_in), x.dtype).at[:n].set(x)

  deg_kernel = _make_deg_kernel(n_chunks, n_pad)
  deg_p = deg_kernel(pk2d).reshape(2, NW, n_pad)

  grid = n_pad // RB
  h1, a_in, a_out = pl.pallas_call(
      _prep1_body,
      grid=(grid,),
      in_specs=[
          pl.BlockSpec((NW, RB), lambda i: (0, i)),
          pl.BlockSpec((NW, RB), lambda i: (0, i)),
          pl.BlockSpec((RB, d_in), lambda i: (i, 0)),
          pl.BlockSpec((d_in, d_hid), lambda i: (0, 0)),
      ],
      out_specs=[
          pl.BlockSpec((RB, d_hid), lambda i: (i, 0)),
          pl.BlockSpec((RB, 1), lambda i: (i, 0)),
          pl.BlockSpec((RB, 1), lambda i: (i, 0)),
      ],
      out_shape=[
          jax.ShapeDtypeStruct((n_pad, d_hid), jnp.float32),
          jax.ShapeDtypeStruct((n_pad, 1), jnp.float32),
          jax.ShapeDtypeStruct((n_pad, 1), jnp.float32),
      ],
  )(deg_p[0], deg_p[1], x_pad, W0)

  src64 = src_p.reshape(e_pad // 64, 64)
  dst64 = dst_p.reshape(e_pad // 64, 64)
  agg_h = _make_agg_kernel(d_hid, e_w, n_pad, 64, 4, 2)
  p1 = agg_h(h1, src64, dst64)

  h2 = pl.pallas_call(
      _mid_body,
      grid=(grid,),
      in_specs=[
          pl.BlockSpec((NC, RB, d_hid), lambda i: (0, i, 0)),
          pl.BlockSpec((RB, 1), lambda i: (i, 0)),
          pl.BlockSpec((RB, 1), lambda i: (i, 0)),
          pl.BlockSpec((d_hid, d_out), lambda i: (0, 0)),
      ],
      out_specs=pl.BlockSpec((RB, d_out), lambda i: (i, 0)),
      out_shape=jax.ShapeDtypeStruct((n_pad, d_out), jnp.float32),
  )(p1, a_in, a_out, W1)

  src128 = src_p.reshape(e_pad // CH, CH)
  dst128 = dst_p.reshape(e_pad // CH, CH)
  agg_o = _make_agg_kernel(d_out, e_w, n_pad, CH, 8, 1)
  p2 = agg_o(h2, src128, dst128)

  out = pl.pallas_call(
      _final_body,
      grid=(grid,),
      in_specs=[
          pl.BlockSpec((NC, RB, d_out), lambda i: (0, i, 0)),
          pl.BlockSpec((RB, 1), lambda i: (i, 0)),
      ],
      out_specs=pl.BlockSpec((RB, d_out), lambda i: (i, 0)),
      out_shape=jax.ShapeDtypeStruct((n_pad, d_out), jnp.float32),
  )(p2, a_in)

  return out[:n]
